# Initial kernel scaffold; baseline (speedup 1.0000x reference)
import jax
import jax.numpy as jnp
from jax.experimental import pallas as pl


def kernel(tile_ids, table):
    raise NotImplementedError("write your pallas kernel here")


# speedup vs baseline: 5.1184x; 5.1184x over previous
import jax
import jax.numpy as jnp
from jax import lax
from jax.experimental import pallas as pl
from jax.experimental.pallas import tpu as pltpu
from jax.experimental.pallas import tpu_sc as plsc

NUM_ROWS_TABLE = 9
NUM_ROWS_OUT = 8
D = 8197120
NW = 32
QUARTERS = NW // NUM_ROWS_OUT
QUARTER = D // QUARTERS
CW = 61440
NFULL = QUARTER // CW
TAIL = QUARTER - NFULL * CW
assert QUARTER % 128 == 0 and CW % 128 == 0 and TAIL % 128 == 0


def _body(ids_hbm, table_hbm, out_hbm, ids_v, idx_v, b0, b1,
          si0, si1, so0, so1):
    w = lax.axis_index("c") * 16 + lax.axis_index("s")
    r = w // QUARTERS
    q = w % QUARTERS

    pltpu.sync_copy(ids_hbm, ids_v)
    rvec = jnp.full((16,), 0, jnp.int32) + r
    idx_v[...] = plsc.load_gather(ids_v, [rvec])

    col0 = pl.multiple_of(q * QUARTER, 128)

    bufs = (b0, b1)
    sin = (si0, si1)
    sout = (so0, so1)
    sizes = [CW] * NFULL + ([TAIL] if TAIL else [])
    offs = [k * CW for k in range(len(sizes))]

    def start_in(k):
        return pltpu.async_copy(
            table_hbm.at[idx_v.at[pl.ds(0, 1)],
                         pl.ds(col0 + offs[k], sizes[k])],
            bufs[k % 2].at[:, pl.ds(0, sizes[k])], sin[k % 2])

    def start_out(k):
        return pltpu.async_copy(
            bufs[k % 2].at[:, pl.ds(0, sizes[k])],
            out_hbm.at[pl.ds(r, 1), pl.ds(col0 + offs[k], sizes[k])],
            sout[k % 2])

    n = len(sizes)
    h_in = {0: start_in(0), 1: start_in(1)}
    h_out = {}
    for k in range(n):
        h_in[k].wait()
        h_out[k] = start_out(k)
        if k + 2 < n:
            h_out[k].wait()
            h_in[k + 2] = start_in(k + 2)
    h_out[n - 2].wait()
    h_out[n - 1].wait()


@jax.jit
def kernel(tile_ids, table):
    mesh = plsc.VectorSubcoreMesh(core_axis_name="c", subcore_axis_name="s")
    run = pl.kernel(
        _body,
        out_type=jax.ShapeDtypeStruct((NUM_ROWS_OUT, D), jnp.float32),
        mesh=mesh,
        compiler_params=pltpu.CompilerParams(needs_layout_passes=False),
        scratch_types=[
            pltpu.VMEM((NUM_ROWS_OUT,), jnp.int32),
            pltpu.VMEM((16,), jnp.int32),
            pltpu.VMEM((1, CW), jnp.float32),
            pltpu.VMEM((1, CW), jnp.float32),
            pltpu.SemaphoreType.DMA,
            pltpu.SemaphoreType.DMA,
            pltpu.SemaphoreType.DMA,
            pltpu.SemaphoreType.DMA,
        ],
    )
    return run(tile_ids.astype(jnp.int32), table)
